# baseline (speedup 1.0000x reference)
import functools

import jax
import jax.numpy as jnp
from jax import lax
from jax.experimental import pallas as pl
from jax.experimental.pallas import tpu as pltpu
from jax.experimental.pallas import tpu_sc as plsc

GAMMA = 0.9
B = 16384
DATA_LEN = 1000000
NW = 32
RANGE = 31264
WSIZE = NW * RANGE
BPW = B // NW
VPB = B // 16
LPW = BPW // 16

_mesh = plsc.VectorSubcoreMesh(core_axis_name="c", subcore_axis_name="s")


def _wid():
    return lax.axis_index("s") * 2 + lax.axis_index("c")


@functools.partial(
    pl.kernel,
    out_type=jax.ShapeDtypeStruct((WSIZE,), jnp.int32),
    mesh=_mesh,
    compiler_params=pltpu.CompilerParams(needs_layout_passes=False),
    scratch_types=[
        pltpu.VMEM((B,), jnp.int32),
        pltpu.VMEM((RANGE,), jnp.int32),
    ],
)
def _winner_kernel(idx_hbm, w_hbm, idxv, wtab):
    wid = _wid()
    lo = wid * RANGE
    pltpu.sync_copy(idx_hbm, idxv)
    lane = lax.iota(jnp.int32, 16)

    def scan_body(r, cnt):
        iv = idxv[pl.ds(r * 16, 16)]
        pos = r * 16 + lane
        m = (iv >= lo) & (iv < lo + RANGE)
        lidx = jnp.where(m, iv - lo, 0)
        plsc.store_scatter(wtab, [lidx], pos, mask=m)
        rv = plsc.load_gather(wtab, [lidx], mask=m)
        m2 = m & (rv < pos)
        plsc.store_scatter(wtab, [lidx], pos, mask=m2)
        return cnt + m2.astype(jnp.int32)

    cntv = lax.fori_loop(0, VPB, scan_body, jnp.zeros((16,), jnp.int32))

    def fix_body(r, cnt):
        iv = idxv[pl.ds(r * 16, 16)]
        pos = r * 16 + lane
        m = (iv >= lo) & (iv < lo + RANGE)
        lidx = jnp.where(m, iv - lo, 0)
        rv = plsc.load_gather(wtab, [lidx], mask=m)
        m2 = m & (rv < pos)
        plsc.store_scatter(wtab, [lidx], pos, mask=m2)
        return cnt + m2.astype(jnp.int32)

    def fix_pass(cntv):
        return lax.fori_loop(0, VPB, fix_body, jnp.zeros((16,), jnp.int32))

    lax.while_loop(lambda c: jnp.sum(c) > 0, fix_pass, cntv)
    pltpu.sync_copy(wtab, w_hbm.at[pl.ds(lo, RANGE)])


@functools.partial(
    pl.kernel,
    out_type=jax.ShapeDtypeStruct((NW * 16,), jnp.float32),
    mesh=_mesh,
    compiler_params=pltpu.CompilerParams(needs_layout_passes=False),
    scratch_types=[
        pltpu.VMEM((BPW,), jnp.int32),
        pltpu.VMEM((BPW,), jnp.int32),
        pltpu.VMEM((BPW,), jnp.float32),
        pltpu.VMEM((BPW,), jnp.float32),
        pltpu.VMEM((BPW,), jnp.float32),
        pltpu.VMEM((BPW,), jnp.float32),
        pltpu.VMEM((BPW,), jnp.float32),
        pltpu.VMEM((BPW,), jnp.float32),
        pltpu.VMEM((BPW,), jnp.int32),
        pltpu.VMEM((32,), jnp.float32),
        pltpu.VMEM((16,), jnp.float32),
        pltpu.SemaphoreType.DMA,
        pltpu.SemaphoreType.DMA,
        pltpu.SemaphoreType.DMA,
        pltpu.SemaphoreType.DMA,
    ],
)
def _loss_kernel(idx_hbm, sn_hbm, sd_hbm, y_hbm, snb_hbm, sdb_hbm, ab_hbm,
                 w_hbm, out_hbm, idxv, wv, gsn, gsd, snw, sdw,
                 snv, sdv, yv, abv, outv, semA, semB, semC, semD):
    wid = _wid()
    base = wid * BPW
    snb = snb_hbm.at[0]
    sdb = sdb_hbm.at[0]
    pltpu.sync_copy(idx_hbm.at[pl.ds(base, BPW)], idxv)

    cps = []
    for j in range(4):
        s = pl.ds(j * 128, 128)
        cps.append(pltpu.async_copy(w_hbm.at[idxv.at[s]], wv.at[s], semA))
        cps.append(pltpu.async_copy(snb.at[idxv.at[s]], gsn.at[s], semB))
        cps.append(pltpu.async_copy(sdb.at[idxv.at[s]], gsd.at[s], semB))
    cps.append(pltpu.async_copy(sn_hbm.at[pl.ds(base, BPW)], snv, semC))
    cps.append(pltpu.async_copy(sd_hbm.at[pl.ds(base, BPW)], sdv, semC))
    cps.append(pltpu.async_copy(y_hbm.at[pl.ds(base, BPW)], yv, semC))
    cps.append(pltpu.async_copy(ab_hbm, abv, semC))
    for j in range(4):
        cps[3 * j].wait()
    cps2 = []
    for j in range(4):
        s = pl.ds(j * 128, 128)
        cps2.append(pltpu.async_copy(sn_hbm.at[wv.at[s]], snw.at[s], semD))
        cps2.append(pltpu.async_copy(sd_hbm.at[wv.at[s]], sdw.at[s], semD))
    for j in range(4):
        cps[3 * j + 1].wait()
        cps[3 * j + 2].wait()
    for cp in cps[12:]:
        cp.wait()
    for cp in cps2:
        cp.wait()

    av = abv[pl.ds(0, 16)]
    bv = abv[pl.ds(16, 16)]
    zero = jnp.zeros((16,), jnp.float32)

    def body(r, accs):
        a0, a1, a2, a3, a4, a5, a6, a7 = accs
        s = pl.ds(r * 16, 16)
        g_sn = gsn[s]
        g_sd = gsd[s]
        s_w = snw[s]
        d_w = sdw[s]
        s_p = snv[s]
        d_p = sdv[s]
        y = yv[s]
        vsn = (1.0 - GAMMA) * g_sn + GAMMA * s_w
        vsd = jnp.maximum((1.0 - GAMMA) * g_sd + GAMMA * d_w, 1e-08)
        rcp = 1.0 / vsd
        z = vsn * rcp
        snd = 1.0 / (1.0 + jnp.exp(-z))
        gsnd = snd * (1.0 - snd)
        gw = gsnd * (rcp * s_p - (vsn * rcp * rcp) * d_p)
        mp = (y == 1).astype(jnp.float32)
        mn = (y == 0).astype(jnp.float32)
        ta = snd - av
        tb = snd - bv
        return (a0 + mp, a1 + mn,
                a2 + mp * (2.0 * ta * gw), a3 + mn * (2.0 * tb * gw),
                a4 + mn * gw, a5 + mp * gw,
                a6 + mp * ta * ta, a7 + mn * tb * tb)

    accs = lax.fori_loop(0, LPW, body, (zero,) * 8)
    lane = lax.iota(jnp.int32, 16)
    ov = jnp.zeros((16,), jnp.float32)
    for k in range(8):
        ov = ov + jnp.where(lane == k, jnp.sum(accs[k]), 0.0)
    outv[...] = ov
    pltpu.sync_copy(outv, out_hbm.at[pl.ds(wid * 16, 16)])


def kernel(sn, sd, y_true, index, sn_buf, sd_buf, a, b, alpha):
    idx = index.reshape(-1).astype(jnp.int32)
    sn_f = sn.reshape(-1)
    sd_f = sd.reshape(-1)
    y = y_true.reshape(-1)
    ab = jnp.concatenate([jnp.broadcast_to(a, (16,)), jnp.broadcast_to(b, (16,))])
    w = _winner_kernel(idx)
    out = _loss_kernel(idx, sn_f, sd_f, y, sn_buf.reshape(1, -1), sd_buf.reshape(1, -1), ab, w)
    s = out.reshape(NW, 16).sum(axis=0)
    n_p = s[0]
    n_n = s[1]
    return (s[2] / n_p + s[3] / n_n
            + alpha[0] * (s[4] / n_n - s[5] / n_p)
            + s[6] / n_p + s[7] / n_n)


# speedup vs baseline: 1.0031x; 1.0031x over previous
import functools

import jax
import jax.numpy as jnp
from jax import lax
from jax.experimental import pallas as pl
from jax.experimental.pallas import tpu as pltpu
from jax.experimental.pallas import tpu_sc as plsc

GAMMA = 0.9
B = 16384
DATA_LEN = 1000000
NW = 32
NW1 = 16
RANGE = 62528
WSIZE = NW1 * RANGE
BPW = B // NW
VPB = B // 16
LPW = BPW // 16

_mesh = plsc.VectorSubcoreMesh(core_axis_name="c", subcore_axis_name="s")


def _wid():
    return lax.axis_index("s") * 2 + lax.axis_index("c")


@functools.partial(
    pl.kernel,
    out_type=jax.ShapeDtypeStruct((WSIZE,), jnp.int32),
    mesh=_mesh,
    compiler_params=pltpu.CompilerParams(needs_layout_passes=False),
    scratch_types=[
        pltpu.VMEM((B,), jnp.int32),
        pltpu.VMEM((RANGE,), jnp.int32),
    ],
)
def _winner_kernel(idx_hbm, w_hbm, idxv, wtab):
    @pl.when(lax.axis_index("c") == 0)
    def _():
        wid = lax.axis_index("s")
        lo = wid * RANGE
        pltpu.sync_copy(idx_hbm, idxv)
        lane = lax.iota(jnp.int32, 16)

        def scan_body(r, cnt):
            iv = idxv[pl.ds(r * 16, 16)]
            pos = r * 16 + lane
            m = (iv >= lo) & (iv < lo + RANGE)
            lidx = jnp.where(m, iv - lo, 0)
            plsc.store_scatter(wtab, [lidx], pos, mask=m)
            rv = plsc.load_gather(wtab, [lidx], mask=m)
            m2 = m & (rv < pos)
            plsc.store_scatter(wtab, [lidx], pos, mask=m2)
            return cnt + m2.astype(jnp.int32)

        cntv = lax.fori_loop(0, VPB, scan_body, jnp.zeros((16,), jnp.int32))

        def fix_body(r, cnt):
            iv = idxv[pl.ds(r * 16, 16)]
            pos = r * 16 + lane
            m = (iv >= lo) & (iv < lo + RANGE)
            lidx = jnp.where(m, iv - lo, 0)
            rv = plsc.load_gather(wtab, [lidx], mask=m)
            m2 = m & (rv < pos)
            plsc.store_scatter(wtab, [lidx], pos, mask=m2)
            return cnt + m2.astype(jnp.int32)

        def fix_pass(cntv):
            return lax.fori_loop(0, VPB, fix_body, jnp.zeros((16,), jnp.int32))

        lax.while_loop(lambda c: jnp.sum(c) > 0, fix_pass, cntv)
        pltpu.sync_copy(wtab, w_hbm.at[pl.ds(lo, RANGE)])


@functools.partial(
    pl.kernel,
    out_type=jax.ShapeDtypeStruct((NW * 16,), jnp.float32),
    mesh=_mesh,
    compiler_params=pltpu.CompilerParams(needs_layout_passes=False),
    scratch_types=[
        pltpu.VMEM((BPW,), jnp.int32),
        pltpu.VMEM((BPW,), jnp.int32),
        pltpu.VMEM((BPW,), jnp.float32),
        pltpu.VMEM((BPW,), jnp.float32),
        pltpu.VMEM((BPW,), jnp.float32),
        pltpu.VMEM((BPW,), jnp.float32),
        pltpu.VMEM((BPW,), jnp.float32),
        pltpu.VMEM((BPW,), jnp.float32),
        pltpu.VMEM((BPW,), jnp.int32),
        pltpu.VMEM((32,), jnp.float32),
        pltpu.VMEM((16,), jnp.float32),
        pltpu.SemaphoreType.DMA,
        pltpu.SemaphoreType.DMA,
        pltpu.SemaphoreType.DMA,
        pltpu.SemaphoreType.DMA,
    ],
)
def _loss_kernel(idx_hbm, sn_hbm, sd_hbm, y_hbm, snb_hbm, sdb_hbm, ab_hbm,
                 w_hbm, out_hbm, idxv, wv, gsn, gsd, snw, sdw,
                 snv, sdv, yv, abv, outv, semA, semB, semC, semD):
    wid = _wid()
    base = wid * BPW
    snb = snb_hbm.at[0]
    sdb = sdb_hbm.at[0]
    pltpu.sync_copy(idx_hbm.at[pl.ds(base, BPW)], idxv)

    cps = []
    for j in range(4):
        s = pl.ds(j * 128, 128)
        cps.append(pltpu.async_copy(w_hbm.at[idxv.at[s]], wv.at[s], semA))
        cps.append(pltpu.async_copy(snb.at[idxv.at[s]], gsn.at[s], semB))
        cps.append(pltpu.async_copy(sdb.at[idxv.at[s]], gsd.at[s], semB))
    cps.append(pltpu.async_copy(sn_hbm.at[pl.ds(base, BPW)], snv, semC))
    cps.append(pltpu.async_copy(sd_hbm.at[pl.ds(base, BPW)], sdv, semC))
    cps.append(pltpu.async_copy(y_hbm.at[pl.ds(base, BPW)], yv, semC))
    cps.append(pltpu.async_copy(ab_hbm, abv, semC))
    for j in range(4):
        cps[3 * j].wait()
    cps2 = []
    for j in range(4):
        s = pl.ds(j * 128, 128)
        cps2.append(pltpu.async_copy(sn_hbm.at[wv.at[s]], snw.at[s], semD))
        cps2.append(pltpu.async_copy(sd_hbm.at[wv.at[s]], sdw.at[s], semD))
    for j in range(4):
        cps[3 * j + 1].wait()
        cps[3 * j + 2].wait()
    for cp in cps[12:]:
        cp.wait()
    for cp in cps2:
        cp.wait()

    av = abv[pl.ds(0, 16)]
    bv = abv[pl.ds(16, 16)]
    zero = jnp.zeros((16,), jnp.float32)

    def body(r, accs):
        a0, a1, a2, a3, a4, a5, a6, a7 = accs
        s = pl.ds(r * 16, 16)
        g_sn = gsn[s]
        g_sd = gsd[s]
        s_w = snw[s]
        d_w = sdw[s]
        s_p = snv[s]
        d_p = sdv[s]
        y = yv[s]
        vsn = (1.0 - GAMMA) * g_sn + GAMMA * s_w
        vsd = jnp.maximum((1.0 - GAMMA) * g_sd + GAMMA * d_w, 1e-08)
        rcp = 1.0 / vsd
        z = vsn * rcp
        snd = 1.0 / (1.0 + jnp.exp(-z))
        gsnd = snd * (1.0 - snd)
        gw = gsnd * (rcp * s_p - (vsn * rcp * rcp) * d_p)
        mp = (y == 1).astype(jnp.float32)
        mn = (y == 0).astype(jnp.float32)
        ta = snd - av
        tb = snd - bv
        return (a0 + mp, a1 + mn,
                a2 + mp * (2.0 * ta * gw), a3 + mn * (2.0 * tb * gw),
                a4 + mn * gw, a5 + mp * gw,
                a6 + mp * ta * ta, a7 + mn * tb * tb)

    accs = lax.fori_loop(0, LPW, body, (zero,) * 8)
    lane = lax.iota(jnp.int32, 16)
    ov = jnp.zeros((16,), jnp.float32)
    for k in range(8):
        ov = ov + jnp.where(lane == k, jnp.sum(accs[k]), 0.0)
    outv[...] = ov
    pltpu.sync_copy(outv, out_hbm.at[pl.ds(wid * 16, 16)])


def kernel(sn, sd, y_true, index, sn_buf, sd_buf, a, b, alpha):
    idx = index.reshape(-1).astype(jnp.int32)
    sn_f = sn.reshape(-1)
    sd_f = sd.reshape(-1)
    y = y_true.reshape(-1)
    ab = jnp.concatenate([jnp.broadcast_to(a, (16,)), jnp.broadcast_to(b, (16,))])
    w = _winner_kernel(idx)
    out = _loss_kernel(idx, sn_f, sd_f, y, sn_buf.reshape(1, -1), sd_buf.reshape(1, -1), ab, w)
    s = out.reshape(NW, 16).sum(axis=0)
    n_p = s[0]
    n_n = s[1]
    return (s[2] / n_p + s[3] / n_n
            + alpha[0] * (s[4] / n_n - s[5] / n_p)
            + s[6] / n_p + s[7] / n_n)


# speedup vs baseline: 1.2128x; 1.2090x over previous
import functools

import jax
import jax.numpy as jnp
from jax import lax
from jax.experimental import pallas as pl
from jax.experimental.pallas import tpu as pltpu
from jax.experimental.pallas import tpu_sc as plsc

GAMMA = 0.9
B = 16384
DATA_LEN = 1000000
NW1 = 16
RANGE = 62528
WSIZE = NW1 * RANGE
BPW = B // NW1
VPB = B // 16
LPW = BPW // 16
NCH = BPW // 128

_mesh = plsc.VectorSubcoreMesh(core_axis_name="c", subcore_axis_name="s")


@functools.partial(
    pl.kernel,
    out_type=(
        jax.ShapeDtypeStruct((WSIZE,), jnp.int32),
        jax.ShapeDtypeStruct((NW1 * 16,), jnp.float32),
        jax.ShapeDtypeStruct((16,), jnp.float32),
    ),
    mesh=_mesh,
    compiler_params=pltpu.CompilerParams(needs_layout_passes=False),
    scratch_types=[
        pltpu.VMEM((B,), jnp.int32),
        pltpu.VMEM((RANGE,), jnp.int32),
        pltpu.VMEM((BPW,), jnp.int32),
        pltpu.VMEM((BPW,), jnp.float32),
        pltpu.VMEM((BPW,), jnp.float32),
        pltpu.VMEM((BPW,), jnp.float32),
        pltpu.VMEM((BPW,), jnp.float32),
        pltpu.VMEM((BPW,), jnp.float32),
        pltpu.VMEM((BPW,), jnp.float32),
        pltpu.VMEM((BPW,), jnp.int32),
        pltpu.VMEM((48,), jnp.float32),
        pltpu.VMEM((NW1 * 16,), jnp.float32),
        pltpu.VMEM((16,), jnp.float32),
        pltpu.SemaphoreType.DMA,
        pltpu.SemaphoreType.DMA,
        pltpu.SemaphoreType.DMA,
        pltpu.SemaphoreType.DMA,
    ],
)
def _fused_kernel(idx_hbm, sn_hbm, sd_hbm, y_hbm, snb_hbm, sdb_hbm, ab_hbm,
                  w_hbm, psum_hbm, out_hbm,
                  idxv, wtab, wv, gsn, gsd, snw, sdw, snv, sdv, yv,
                  abv, sumv, outv, semA, semB, semC, semD):
    @pl.when(lax.axis_index("c") == 0)
    def _():
        wid = lax.axis_index("s")
        lo = wid * RANGE
        pltpu.sync_copy(idx_hbm, idxv)
        lane = lax.iota(jnp.int32, 16)

        def scan_body(r, cnt):
            iv = idxv[pl.ds(r * 16, 16)]
            pos = r * 16 + lane
            m = (iv >= lo) & (iv < lo + RANGE)
            lidx = jnp.where(m, iv - lo, 0)
            plsc.store_scatter(wtab, [lidx], pos, mask=m)
            rv = plsc.load_gather(wtab, [lidx], mask=m)
            m2 = m & (rv < pos)
            plsc.store_scatter(wtab, [lidx], pos, mask=m2)
            return cnt + m2.astype(jnp.int32)

        cntv = lax.fori_loop(0, VPB, scan_body, jnp.zeros((16,), jnp.int32))

        def fix_body(r, cnt):
            iv = idxv[pl.ds(r * 16, 16)]
            pos = r * 16 + lane
            m = (iv >= lo) & (iv < lo + RANGE)
            lidx = jnp.where(m, iv - lo, 0)
            rv = plsc.load_gather(wtab, [lidx], mask=m)
            m2 = m & (rv < pos)
            plsc.store_scatter(wtab, [lidx], pos, mask=m2)
            return cnt + m2.astype(jnp.int32)

        def fix_pass(cntv):
            return lax.fori_loop(0, VPB, fix_body, jnp.zeros((16,), jnp.int32))

        lax.while_loop(lambda c: jnp.sum(c) > 0, fix_pass, cntv)
        pltpu.sync_copy(wtab, w_hbm.at[pl.ds(lo, RANGE)])

        plsc.subcore_barrier()

        base = wid * BPW
        snb = snb_hbm.at[0]
        sdb = sdb_hbm.at[0]
        cps = []
        for j in range(NCH):
            s = pl.ds(base + j * 128, 128)
            d = pl.ds(j * 128, 128)
            cps.append(pltpu.async_copy(w_hbm.at[idxv.at[s]], wv.at[d], semA))
            cps.append(pltpu.async_copy(snb.at[idxv.at[s]], gsn.at[d], semB))
            cps.append(pltpu.async_copy(sdb.at[idxv.at[s]], gsd.at[d], semB))
        cps.append(pltpu.async_copy(sn_hbm.at[pl.ds(base, BPW)], snv, semC))
        cps.append(pltpu.async_copy(sd_hbm.at[pl.ds(base, BPW)], sdv, semC))
        cps.append(pltpu.async_copy(y_hbm.at[pl.ds(base, BPW)], yv, semC))
        cps.append(pltpu.async_copy(ab_hbm, abv, semC))
        for j in range(NCH):
            cps[3 * j].wait()
        cps2 = []
        for j in range(NCH):
            d = pl.ds(j * 128, 128)
            cps2.append(pltpu.async_copy(sn_hbm.at[wv.at[d]], snw.at[d], semD))
            cps2.append(pltpu.async_copy(sd_hbm.at[wv.at[d]], sdw.at[d], semD))
        for j in range(NCH):
            cps[3 * j + 1].wait()
            cps[3 * j + 2].wait()
        for cp in cps[3 * NCH:]:
            cp.wait()
        for cp in cps2:
            cp.wait()

        av = abv[pl.ds(0, 16)]
        bv = abv[pl.ds(16, 16)]
        zero = jnp.zeros((16,), jnp.float32)

        def body(r, accs):
            a0, a1, a2, a3, a4, a5, a6, a7 = accs
            s = pl.ds(r * 16, 16)
            g_sn = gsn[s]
            g_sd = gsd[s]
            s_w = snw[s]
            d_w = sdw[s]
            s_p = snv[s]
            d_p = sdv[s]
            y = yv[s]
            vsn = (1.0 - GAMMA) * g_sn + GAMMA * s_w
            vsd = jnp.maximum((1.0 - GAMMA) * g_sd + GAMMA * d_w, 1e-08)
            rcp = 1.0 / vsd
            z = vsn * rcp
            snd = 1.0 / (1.0 + jnp.exp(-z))
            gsnd = snd * (1.0 - snd)
            gw = gsnd * (rcp * s_p - (vsn * rcp * rcp) * d_p)
            mp = (y == 1).astype(jnp.float32)
            mn = (y == 0).astype(jnp.float32)
            ta = snd - av
            tb = snd - bv
            return (a0 + mp, a1 + mn,
                    a2 + mp * (2.0 * ta * gw), a3 + mn * (2.0 * tb * gw),
                    a4 + mn * gw, a5 + mp * gw,
                    a6 + mp * ta * ta, a7 + mn * tb * tb)

        accs = lax.fori_loop(0, LPW, body, (zero,) * 8)
        ov = jnp.zeros((16,), jnp.float32)
        for k in range(8):
            ov = ov + jnp.where(lane == k, jnp.sum(accs[k]), 0.0)
        outv[...] = ov
        pltpu.sync_copy(outv, psum_hbm.at[pl.ds(wid * 16, 16)])

        plsc.subcore_barrier()

        @pl.when(wid == 0)
        def _():
            pltpu.sync_copy(psum_hbm, sumv)

            def red(r, t):
                return t + sumv[pl.ds(r * 16, 16)]

            t = lax.fori_loop(0, NW1, red, jnp.zeros((16,), jnp.float32))

            def lane_sum(vec, k):
                return jnp.sum(jnp.where(lane == k, vec, 0.0))

            s0 = lane_sum(t, 0)
            s1 = lane_sum(t, 1)
            s2 = lane_sum(t, 2)
            s3 = lane_sum(t, 3)
            s4 = lane_sum(t, 4)
            s5 = lane_sum(t, 5)
            s6 = lane_sum(t, 6)
            s7 = lane_sum(t, 7)
            alpha = lane_sum(abv[pl.ds(32, 16)], 0)
            num = jnp.where(lane == 0, s2,
                  jnp.where(lane == 1, s3,
                  jnp.where(lane == 2, alpha * s4,
                  jnp.where(lane == 3, -alpha * s5,
                  jnp.where(lane == 4, s6,
                  jnp.where(lane == 5, s7, 0.0))))))
            den = jnp.where((lane == 0) | (lane == 3) | (lane == 4), s0,
                  jnp.where((lane == 1) | (lane == 2) | (lane == 5), s1, 1.0))
            loss = jnp.sum(num / den)
            outv[...] = jnp.where(lane == 0, loss, 0.0)
            pltpu.sync_copy(outv, out_hbm)


def kernel(sn, sd, y_true, index, sn_buf, sd_buf, a, b, alpha):
    idx = index.reshape(-1).astype(jnp.int32)
    sn_f = sn.reshape(-1)
    sd_f = sd.reshape(-1)
    y = y_true.reshape(-1)
    ab = jnp.concatenate([
        jnp.broadcast_to(a, (16,)),
        jnp.broadcast_to(b, (16,)),
        jnp.broadcast_to(alpha, (16,)),
    ])
    _, _, out = _fused_kernel(idx, sn_f, sd_f, y,
                              sn_buf.reshape(1, -1), sd_buf.reshape(1, -1), ab)
    return out[0]
